# baseline (speedup 1.0000x reference)
import jax
import jax.numpy as jnp
from jax.experimental import pallas as pl
from jax.experimental.pallas import tpu as pltpu

_NSEG = 16
_N = 32768
_B = 16384
_NB = _N // _B
_CH = 512
_NCH = _B // _CH
_NEG = float("-inf")


def _mlp(xb, W1_ref, b1_ref, W2_ref, b2_ref):
    h = jnp.maximum(
        jnp.dot(xb.astype(jnp.bfloat16), W1_ref[...].astype(jnp.bfloat16),
                preferred_element_type=jnp.float32) + b1_ref[...], 0.0)
    return (jnp.dot(h.astype(jnp.bfloat16), W2_ref[...].astype(jnp.bfloat16),
                    preferred_element_type=jnp.float32) + b2_ref[...])


def _fused(x_ref, pos_ref, batch_ref, W1_ref, b1_ref, W2_ref, b2_ref,
           xout_ref, posout_ref):
    i = pl.program_id(0)

    @pl.when(i == 0)
    def _init():
        xout_ref[...] = jnp.full(xout_ref.shape, _NEG, jnp.float32)
        posout_ref[...] = jnp.full(posout_ref.shape, _NEG, jnp.float32)

    h = _mlp(x_ref[...], W1_ref, b1_ref, W2_ref, b2_ref)
    pos_blk = pos_ref[...]
    bm = batch_ref[0]

    C = jnp.max(h.reshape(_NCH, _CH, 256), axis=1)
    Cp = jnp.max(pos_blk.reshape(_NCH, _CH, 3), axis=1)

    firsts = bm[:, 0:1]
    lasts = bm[:, _CH - 1:_CH]
    siota = jax.lax.broadcasted_iota(jnp.int32, (_NCH, _NSEG), 1)
    inside = (firsts == siota) & (lasts == siota)
    cpen = jnp.where(inside, 0.0, _NEG)

    for s in range(_NSEG):
        col = cpen[:, s:s + 1]
        cand = jnp.max(C + col, axis=0, keepdims=True)
        xout_ref[s:s + 1, :] = jnp.maximum(xout_ref[s:s + 1, :], cand)
        pcand = jnp.max(Cp + col, axis=0, keepdims=True)
        posout_ref[s:s + 1, :] = jnp.maximum(posout_ref[s:s + 1, :], pcand)

    lo = bm[0, 0]
    hi = bm[_NCH - 1, _CH - 1]
    kiota = jax.lax.broadcasted_iota(jnp.int32, (_CH, 1), 0)

    def _count_lt(s):
        return jnp.sum((bm < s).astype(jnp.int32))

    def _boundary(j, carry):
        s_r = lo + 1 + j
        r = _count_lt(s_r)
        coff = (r // _CH) * _CH
        hs = _mlp(x_ref[pl.ds(coff, _CH), :], W1_ref, b1_ref, W2_ref, b2_ref)
        ps = pos_ref[pl.ds(coff, _CH), :]

        def _fold(s, a, b):
            rowpen = jnp.where((kiota >= a - coff) & (kiota < b - coff),
                               0.0, _NEG)
            cand = jnp.max(hs + rowpen, axis=0, keepdims=True)
            xout_ref[pl.ds(s, 1), :] = jnp.maximum(xout_ref[pl.ds(s, 1), :],
                                                   cand)
            pcand = jnp.max(ps + rowpen, axis=0, keepdims=True)
            posout_ref[pl.ds(s, 1), :] = jnp.maximum(
                posout_ref[pl.ds(s, 1), :], pcand)

        a_left = _count_lt(s_r - 1)
        b_right = _count_lt(s_r + 1)
        _fold(s_r - 1, a_left, r)
        _fold(s_r, r, b_right)
        return carry

    jax.lax.fori_loop(0, hi - lo, _boundary, 0)


def kernel(x, pos, batch, W1, b1, W2, b2):
    grid_spec = pl.GridSpec(
        grid=(_NB,),
        in_specs=[
            pl.BlockSpec((_B, 128), lambda i: (i, 0)),
            pl.BlockSpec((_B, 3), lambda i: (i, 0)),
            pl.BlockSpec((1, _NCH, _CH), lambda i: (i, 0, 0)),
            pl.BlockSpec((128, 128), lambda i: (0, 0)),
            pl.BlockSpec((1, 128), lambda i: (0, 0)),
            pl.BlockSpec((128, 256), lambda i: (0, 0)),
            pl.BlockSpec((1, 256), lambda i: (0, 0)),
        ],
        out_specs=[
            pl.BlockSpec((_NSEG, 256), lambda i: (0, 0)),
            pl.BlockSpec((_NSEG, 3), lambda i: (0, 0)),
        ],
    )
    x_out, pos_out = pl.pallas_call(
        _fused,
        grid_spec=grid_spec,
        out_shape=[
            jax.ShapeDtypeStruct((_NSEG, 256), jnp.float32),
            jax.ShapeDtypeStruct((_NSEG, 3), jnp.float32),
        ],
        compiler_params=pltpu.CompilerParams(
            dimension_semantics=("arbitrary",)),
    )(x, pos, batch.reshape(_NB, _NCH, _CH), W1, b1.reshape(1, 128), W2,
      b2.reshape(1, 256))
    batch_out = jnp.arange(_NSEG, dtype=jnp.int32)
    return (x_out, pos_out, batch_out)


# speedup vs baseline: 1.0498x; 1.0498x over previous
import jax
import jax.numpy as jnp
from jax.experimental import pallas as pl
from jax.experimental.pallas import tpu as pltpu

_NSEG = 16
_N = 32768
_B = 8192
_NB = _N // _B
_CH = 512
_NCH = _B // _CH
_NEG = float("-inf")


def _mlp(xb, W1_ref, b1_ref, W2_ref, b2_ref):
    h = jnp.maximum(
        jnp.dot(xb.astype(jnp.bfloat16), W1_ref[...].astype(jnp.bfloat16),
                preferred_element_type=jnp.float32) + b1_ref[...], 0.0)
    return (jnp.dot(h.astype(jnp.bfloat16), W2_ref[...].astype(jnp.bfloat16),
                    preferred_element_type=jnp.float32) + b2_ref[...])


def _fused(x_ref, pos_ref, batch_ref, W1_ref, b1_ref, W2_ref, b2_ref,
           xout_ref, posout_ref):
    i = pl.program_id(0)

    @pl.when(i == 0)
    def _init():
        xout_ref[...] = jnp.full(xout_ref.shape, _NEG, jnp.float32)
        posout_ref[...] = jnp.full(posout_ref.shape, _NEG, jnp.float32)

    h = _mlp(x_ref[...], W1_ref, b1_ref, W2_ref, b2_ref)
    pos_blk = pos_ref[...]
    bm = batch_ref[0]

    C = jnp.max(h.reshape(_NCH, _CH, 256), axis=1)
    Cp = jnp.max(pos_blk.reshape(_NCH, _CH, 3), axis=1)

    firsts = bm[:, 0:1]
    lasts = bm[:, _CH - 1:_CH]
    siota = jax.lax.broadcasted_iota(jnp.int32, (_NCH, _NSEG), 1)
    inside = (firsts == siota) & (lasts == siota)
    cpen = jnp.where(inside, 0.0, _NEG)

    for s in range(_NSEG):
        col = cpen[:, s:s + 1]
        cand = jnp.max(C + col, axis=0, keepdims=True)
        xout_ref[s:s + 1, :] = jnp.maximum(xout_ref[s:s + 1, :], cand)
        pcand = jnp.max(Cp + col, axis=0, keepdims=True)
        posout_ref[s:s + 1, :] = jnp.maximum(posout_ref[s:s + 1, :], pcand)

    lo = bm[0, 0]
    hi = bm[_NCH - 1, _CH - 1]
    kiota = jax.lax.broadcasted_iota(jnp.int32, (_CH, 1), 0)

    def _count_lt(s):
        return jnp.sum((bm < s).astype(jnp.int32))

    def _boundary(j, carry):
        s_r = lo + 1 + j
        r = _count_lt(s_r)
        coff = (r // _CH) * _CH
        hs = _mlp(x_ref[pl.ds(coff, _CH), :], W1_ref, b1_ref, W2_ref, b2_ref)
        ps = pos_ref[pl.ds(coff, _CH), :]

        def _fold(s, a, b):
            rowpen = jnp.where((kiota >= a - coff) & (kiota < b - coff),
                               0.0, _NEG)
            cand = jnp.max(hs + rowpen, axis=0, keepdims=True)
            xout_ref[pl.ds(s, 1), :] = jnp.maximum(xout_ref[pl.ds(s, 1), :],
                                                   cand)
            pcand = jnp.max(ps + rowpen, axis=0, keepdims=True)
            posout_ref[pl.ds(s, 1), :] = jnp.maximum(
                posout_ref[pl.ds(s, 1), :], pcand)

        a_left = _count_lt(s_r - 1)
        b_right = _count_lt(s_r + 1)
        _fold(s_r - 1, a_left, r)
        _fold(s_r, r, b_right)
        return carry

    jax.lax.fori_loop(0, hi - lo, _boundary, 0)


def kernel(x, pos, batch, W1, b1, W2, b2):
    grid_spec = pl.GridSpec(
        grid=(_NB,),
        in_specs=[
            pl.BlockSpec((_B, 128), lambda i: (i, 0)),
            pl.BlockSpec((_B, 3), lambda i: (i, 0)),
            pl.BlockSpec((1, _NCH, _CH), lambda i: (i, 0, 0)),
            pl.BlockSpec((128, 128), lambda i: (0, 0)),
            pl.BlockSpec((1, 128), lambda i: (0, 0)),
            pl.BlockSpec((128, 256), lambda i: (0, 0)),
            pl.BlockSpec((1, 256), lambda i: (0, 0)),
        ],
        out_specs=[
            pl.BlockSpec((_NSEG, 256), lambda i: (0, 0)),
            pl.BlockSpec((_NSEG, 3), lambda i: (0, 0)),
        ],
    )
    x_out, pos_out = pl.pallas_call(
        _fused,
        grid_spec=grid_spec,
        out_shape=[
            jax.ShapeDtypeStruct((_NSEG, 256), jnp.float32),
            jax.ShapeDtypeStruct((_NSEG, 3), jnp.float32),
        ],
        compiler_params=pltpu.CompilerParams(
            dimension_semantics=("arbitrary",)),
    )(x, pos, batch.reshape(_NB, _NCH, _CH), W1, b1.reshape(1, 128), W2,
      b2.reshape(1, 256))
    batch_out = jnp.arange(_NSEG, dtype=jnp.int32)
    return (x_out, pos_out, batch_out)
